# baseline (speedup 1.0000x reference)
import dataclasses
import functools

import jax
import jax.numpy as jnp
from jax import lax
from jax.experimental import pallas as pl
from jax.experimental.pallas import tpu as pltpu
from jax.experimental.pallas import tpu_sc as plsc

_B, _N, _D = 16384, 7, 256
_C = 64
_MR = 72
_NC, _NS, _L = 2, 16, 16
_NW = _NC * _NS
_RSC = 512
_BPW = _RSC // _NW
_TCBLK = 3968


def _table_body(rank_ref, suit_ref, card_ref, w_ref):
    row = lax.broadcasted_iota(jnp.int32, (_C, 1), 0)
    valid = row < 52
    oh_r = jnp.where(
        (row % 13 == lax.broadcasted_iota(jnp.int32, (_C, 16), 1)) & valid,
        1.0, 0.0)
    oh_s = jnp.where(
        (row // 13 == lax.broadcasted_iota(jnp.int32, (_C, 8), 1)) & valid,
        1.0, 0.0)
    rank_pad = jnp.concatenate(
        [rank_ref[...], jnp.zeros((3, _D), jnp.float32)], axis=0)
    suit_pad = jnp.concatenate(
        [suit_ref[...], jnp.zeros((4, _D), jnp.float32)], axis=0)
    card_pad = jnp.concatenate(
        [card_ref[...], jnp.zeros((12, _D), jnp.float32)], axis=0)
    t = (
        lax.dot_general(oh_r, rank_pad, (((1,), (0,)), ((), ())),
                        preferred_element_type=jnp.float32)
        + lax.dot_general(oh_s, suit_pad, (((1,), (0,)), ((), ())),
                          preferred_element_type=jnp.float32)
        + card_pad
    )
    return lax.dot_general(t, w_ref[...], (((1,), (1,)), ((), ())),
                           preferred_element_type=jnp.float32)


def _table_kernel(rank_ref, suit_ref, card_ref, w_ref, b_ref, m_ref):
    m = _table_body(rank_ref, suit_ref, card_ref, w_ref)
    m_ref[...] = jnp.concatenate(
        [m, b_ref[...], jnp.zeros((_MR - _C - 1, _D), jnp.float32)], axis=0)


def _build_table(rank_emb, suit_emb, card_emb, W, b):
    return pl.pallas_call(
        _table_kernel,
        out_shape=jax.ShapeDtypeStruct((_MR, _D), jnp.float32),
    )(rank_emb, suit_emb, card_emb, W, b.reshape(1, _D))


def _sc_body(cards_hbm, m_hbm, out_hbm, m_v, cards_v, out_v):
    c = lax.axis_index("c")
    s = lax.axis_index("s")
    wid = s * _NC + c
    pltpu.sync_copy(m_hbm, m_v)
    pltpu.sync_copy(cards_hbm.at[wid], cards_v)
    lane = lax.iota(jnp.int32, _L)

    pre = [cards_v[n, pl.ds(0, _L)] * _D for n in range(_N + 1)]
    srow = lane * _D

    @plsc.parallel_loop(0, _D, unroll=4, carry=jnp.zeros((_L,), jnp.int32))
    def _col(d, dv):
        g0 = plsc.load_gather(m_v, [pre[0] + dv])
        g1 = plsc.load_gather(m_v, [pre[1] + dv])
        g2 = plsc.load_gather(m_v, [pre[2] + dv])
        g3 = plsc.load_gather(m_v, [pre[3] + dv])
        g4 = plsc.load_gather(m_v, [pre[4] + dv])
        g5 = plsc.load_gather(m_v, [pre[5] + dv])
        g6 = plsc.load_gather(m_v, [pre[6] + dv])
        g7 = plsc.load_gather(m_v, [pre[7] + dv])
        acc = ((g0 + g1) + (g2 + g3)) + ((g4 + g5) + (g6 + g7))
        acc = jnp.maximum(acc, 0.0)
        plsc.store_scatter(out_v, [srow + dv], acc)
        return dv + 1

    pltpu.sync_copy(out_v, out_hbm.at[pl.ds(wid * _BPW * _D, _BPW * _D)])


def _sc_call(cards_sc, m_pad):
    mesh = plsc.VectorSubcoreMesh(core_axis_name="c", subcore_axis_name="s")
    cp = pltpu.CompilerParams()
    if "needs_layout_passes" in pltpu.CompilerParams.__dataclass_fields__:
        cp = dataclasses.replace(cp, needs_layout_passes=False)
    run = pl.kernel(
        _sc_body,
        mesh=mesh,
        compiler_params=cp,
        out_type=jax.ShapeDtypeStruct((_RSC * _D,), jnp.float32),
        scratch_types=[
            pltpu.VMEM((_MR * _D,), jnp.float32),
            pltpu.VMEM((_N + 1, _BPW), jnp.int32),
            pltpu.VMEM((_BPW * _D,), jnp.float32),
        ],
    )
    return run(cards_sc, m_pad.reshape(-1))


def _tc_kernel(cards_ref, rank_ref, suit_ref, card_ref, w_ref, b_ref,
               out_ref, m_scr):
    @pl.when(pl.program_id(0) == 0)
    def _build():
        m_scr[...] = _table_body(rank_ref, suit_ref, card_ref,
                                 w_ref).astype(jnp.bfloat16)

    cards = cards_ref[...].astype(jnp.bfloat16)
    bins = lax.broadcasted_iota(jnp.int32, (_TCBLK, _C), 1).astype(jnp.bfloat16)
    counts = jnp.zeros((_TCBLK, _C), jnp.bfloat16)
    for n in range(_N):
        counts += jnp.where(cards[:, n:n + 1] == bins,
                            jnp.bfloat16(1.0), jnp.bfloat16(0.0))
    acc = lax.dot_general(
        counts, m_scr[...], (((1,), (0,)), ((), ())),
        preferred_element_type=jnp.float32)
    out_ref[...] = jnp.maximum(acc + b_ref[...], 0.0)


def _tc_call(cards_tc, rank_emb, suit_emb, card_emb, W, b):
    n_rows = _B - _RSC
    grid = (n_rows // _TCBLK,)
    return pl.pallas_call(
        _tc_kernel,
        grid=grid,
        in_specs=[
            pl.BlockSpec((_TCBLK, _N), lambda i: (i, 0)),
            pl.BlockSpec((13, _D), lambda i: (0, 0)),
            pl.BlockSpec((4, _D), lambda i: (0, 0)),
            pl.BlockSpec((52, _D), lambda i: (0, 0)),
            pl.BlockSpec((_D, _D), lambda i: (0, 0)),
            pl.BlockSpec((1, _D), lambda i: (0, 0)),
        ],
        out_specs=pl.BlockSpec((_TCBLK, _D), lambda i: (i, 0)),
        out_shape=jax.ShapeDtypeStruct((n_rows, _D), jnp.float32),
        scratch_shapes=[pltpu.VMEM((_C, _D), jnp.bfloat16)],
    )(cards_tc, rank_emb, suit_emb, card_emb, W, b.reshape(1, _D))


def kernel(cards, rank_emb, suit_emb, card_emb, W, b):
    m_pad = _build_table(rank_emb, suit_emb, card_emb, W, b)
    cards_t = cards[:_RSC].reshape(_NW, _BPW, _N).transpose(0, 2, 1)
    bias_row = jnp.full((_NW, 1, _BPW), _C, jnp.int32)
    cards_sc = jnp.concatenate([cards_t, bias_row], axis=1)
    sc_out = _sc_call(cards_sc, m_pad).reshape(_RSC, _D)
    tc_out = _tc_call(cards[_RSC:], rank_emb, suit_emb, card_emb, W, b)
    return jnp.concatenate([sc_out, tc_out], axis=0)


# speedup vs baseline: 1.2757x; 1.2757x over previous
import dataclasses
import functools

import jax
import jax.numpy as jnp
from jax import lax
from jax.experimental import pallas as pl
from jax.experimental.pallas import tpu as pltpu
from jax.experimental.pallas import tpu_sc as plsc

_B, _N, _D = 16384, 7, 256
_C = 64
_MR = 72
_NC, _NS, _L = 2, 16, 16
_NW = _NC * _NS
_RSC = 512
_BPW = _RSC // _NW
_TCBLK = 4096


def _table_body(rank_ref, suit_ref, card_ref, w_ref):
    row = lax.broadcasted_iota(jnp.int32, (_C, 1), 0)
    valid = row < 52
    oh_r = jnp.where(
        (row % 13 == lax.broadcasted_iota(jnp.int32, (_C, 16), 1)) & valid,
        1.0, 0.0)
    oh_s = jnp.where(
        (row // 13 == lax.broadcasted_iota(jnp.int32, (_C, 8), 1)) & valid,
        1.0, 0.0)
    rank_pad = jnp.concatenate(
        [rank_ref[...], jnp.zeros((3, _D), jnp.float32)], axis=0)
    suit_pad = jnp.concatenate(
        [suit_ref[...], jnp.zeros((4, _D), jnp.float32)], axis=0)
    card_pad = jnp.concatenate(
        [card_ref[...], jnp.zeros((12, _D), jnp.float32)], axis=0)
    t = (
        lax.dot_general(oh_r, rank_pad, (((1,), (0,)), ((), ())),
                        preferred_element_type=jnp.float32)
        + lax.dot_general(oh_s, suit_pad, (((1,), (0,)), ((), ())),
                          preferred_element_type=jnp.float32)
        + card_pad
    )
    return lax.dot_general(t, w_ref[...], (((1,), (1,)), ((), ())),
                           preferred_element_type=jnp.float32)


def _table_kernel(rank_ref, suit_ref, card_ref, w_ref, b_ref, m_ref):
    m = _table_body(rank_ref, suit_ref, card_ref, w_ref)
    m_ref[...] = jnp.concatenate(
        [m, b_ref[...], jnp.zeros((_MR - _C - 1, _D), jnp.float32)], axis=0)


def _build_table(rank_emb, suit_emb, card_emb, W, b):
    return pl.pallas_call(
        _table_kernel,
        out_shape=jax.ShapeDtypeStruct((_MR, _D), jnp.float32),
    )(rank_emb, suit_emb, card_emb, W, b.reshape(1, _D))


def _sc_body(cards_hbm, m_hbm, out_hbm, m_v, cards_v, out_v):
    c = lax.axis_index("c")
    s = lax.axis_index("s")
    wid = s * _NC + c
    pltpu.sync_copy(m_hbm, m_v)
    pltpu.sync_copy(cards_hbm.at[wid], cards_v)
    lane = lax.iota(jnp.int32, _L)

    pre = [cards_v[n, pl.ds(0, _L)] * _D for n in range(_N + 1)]
    srow = lane * _D

    @plsc.parallel_loop(0, _D, unroll=4, carry=jnp.zeros((_L,), jnp.int32))
    def _col(d, dv):
        g0 = plsc.load_gather(m_v, [pre[0] + dv])
        g1 = plsc.load_gather(m_v, [pre[1] + dv])
        g2 = plsc.load_gather(m_v, [pre[2] + dv])
        g3 = plsc.load_gather(m_v, [pre[3] + dv])
        g4 = plsc.load_gather(m_v, [pre[4] + dv])
        g5 = plsc.load_gather(m_v, [pre[5] + dv])
        g6 = plsc.load_gather(m_v, [pre[6] + dv])
        g7 = plsc.load_gather(m_v, [pre[7] + dv])
        acc = ((g0 + g1) + (g2 + g3)) + ((g4 + g5) + (g6 + g7))
        acc = jnp.maximum(acc, 0.0)
        plsc.store_scatter(out_v, [srow + dv], acc)
        return dv + 1

    pltpu.sync_copy(out_v, out_hbm.at[pl.ds(wid * _BPW * _D, _BPW * _D)])


def _sc_call(cards_sc, m_pad):
    mesh = plsc.VectorSubcoreMesh(core_axis_name="c", subcore_axis_name="s")
    cp = pltpu.CompilerParams()
    if "needs_layout_passes" in pltpu.CompilerParams.__dataclass_fields__:
        cp = dataclasses.replace(cp, needs_layout_passes=False)
    run = pl.kernel(
        _sc_body,
        mesh=mesh,
        compiler_params=cp,
        out_type=jax.ShapeDtypeStruct((_RSC * _D,), jnp.float32),
        scratch_types=[
            pltpu.VMEM((_MR * _D,), jnp.float32),
            pltpu.VMEM((_N + 1, _BPW), jnp.int32),
            pltpu.VMEM((_BPW * _D,), jnp.float32),
        ],
    )
    return run(cards_sc, m_pad.reshape(-1))


def _tc_kernel(cards_ref, rank_ref, suit_ref, card_ref, w_ref, b_ref,
               out_ref, m_scr):
    @pl.when(pl.program_id(0) == 0)
    def _build():
        m_scr[...] = _table_body(rank_ref, suit_ref, card_ref,
                                 w_ref).astype(jnp.bfloat16)

    cards = cards_ref[...].astype(jnp.bfloat16)
    bins = lax.broadcasted_iota(jnp.int32, (_TCBLK, _C), 1).astype(jnp.bfloat16)
    counts = jnp.zeros((_TCBLK, _C), jnp.bfloat16)
    for n in range(_N):
        counts += jnp.where(cards[:, n:n + 1] == bins,
                            jnp.bfloat16(1.0), jnp.bfloat16(0.0))
    acc = lax.dot_general(
        counts, m_scr[...], (((1,), (0,)), ((), ())),
        preferred_element_type=jnp.float32)
    out_ref[...] = jnp.maximum(acc + b_ref[...], 0.0)


def _tc_call(cards_tc, rank_emb, suit_emb, card_emb, W, b):
    n_rows = cards_tc.shape[0]
    grid = (n_rows // _TCBLK,)
    return pl.pallas_call(
        _tc_kernel,
        grid=grid,
        in_specs=[
            pl.BlockSpec((_TCBLK, _N), lambda i: (i, 0)),
            pl.BlockSpec((13, _D), lambda i: (0, 0)),
            pl.BlockSpec((4, _D), lambda i: (0, 0)),
            pl.BlockSpec((52, _D), lambda i: (0, 0)),
            pl.BlockSpec((_D, _D), lambda i: (0, 0)),
            pl.BlockSpec((1, _D), lambda i: (0, 0)),
        ],
        out_specs=pl.BlockSpec((_TCBLK, _D), lambda i: (i, 0)),
        out_shape=jax.ShapeDtypeStruct((n_rows, _D), jnp.float32),
        scratch_shapes=[pltpu.VMEM((_C, _D), jnp.bfloat16)],
    )(cards_tc, rank_emb, suit_emb, card_emb, W, b.reshape(1, _D))


def kernel(cards, rank_emb, suit_emb, card_emb, W, b):
    tc_out = _tc_call(cards, rank_emb, suit_emb, card_emb, W, b)
    m_pad = _build_table(rank_emb, suit_emb, card_emb, W, b)
    cards_t = cards[:_RSC].reshape(_NW, _BPW, _N).transpose(0, 2, 1)
    bias_row = jnp.full((_NW, 1, _BPW), _C, jnp.int32)
    cards_sc = jnp.concatenate([cards_t, bias_row], axis=1)
    sc_out = _sc_call(cards_sc, m_pad).reshape(_RSC, _D)
    return lax.dynamic_update_slice(tc_out, sc_out, (0, 0))


# speedup vs baseline: 1.3264x; 1.0397x over previous
import dataclasses
import functools

import jax
import jax.numpy as jnp
from jax import lax
from jax.experimental import pallas as pl
from jax.experimental.pallas import tpu as pltpu
from jax.experimental.pallas import tpu_sc as plsc

_B, _N, _D = 16384, 7, 256
_C = 64
_MR = 72
_NC, _NS, _L = 2, 16, 16
_NW = _NC * _NS
_RSC = 512
_BPW = _RSC // _NW
_TCBLK = 4096


def _table_body(rank_ref, suit_ref, card_ref, w_ref):
    row = lax.broadcasted_iota(jnp.int32, (_C, 1), 0)
    valid = row < 52
    oh_r = jnp.where(
        (row % 13 == lax.broadcasted_iota(jnp.int32, (_C, 16), 1)) & valid,
        1.0, 0.0)
    oh_s = jnp.where(
        (row // 13 == lax.broadcasted_iota(jnp.int32, (_C, 8), 1)) & valid,
        1.0, 0.0)
    rank_pad = jnp.concatenate(
        [rank_ref[...], jnp.zeros((3, _D), jnp.float32)], axis=0)
    suit_pad = jnp.concatenate(
        [suit_ref[...], jnp.zeros((4, _D), jnp.float32)], axis=0)
    card_pad = jnp.concatenate(
        [card_ref[...], jnp.zeros((12, _D), jnp.float32)], axis=0)
    t = (
        lax.dot_general(oh_r, rank_pad, (((1,), (0,)), ((), ())),
                        preferred_element_type=jnp.float32)
        + lax.dot_general(oh_s, suit_pad, (((1,), (0,)), ((), ())),
                          preferred_element_type=jnp.float32)
        + card_pad
    )
    return lax.dot_general(t, w_ref[...], (((1,), (1,)), ((), ())),
                           preferred_element_type=jnp.float32)


def _table_kernel(rank_ref, suit_ref, card_ref, w_ref, b_ref, m_ref):
    m = _table_body(rank_ref, suit_ref, card_ref, w_ref)
    m_ref[...] = jnp.concatenate(
        [m, b_ref[...], jnp.zeros((_MR - _C - 1, _D), jnp.float32)], axis=0)


def _build_table(rank_emb, suit_emb, card_emb, W, b):
    return pl.pallas_call(
        _table_kernel,
        out_shape=jax.ShapeDtypeStruct((_MR, _D), jnp.float32),
    )(rank_emb, suit_emb, card_emb, W, b.reshape(1, _D))


_DP2 = _D // 2


def _sc_body(cards_hbm, m_hbm, out_hbm, m_v, cards_v, out_v):
    c = lax.axis_index("c")
    s = lax.axis_index("s")
    wid = s * _NC + c
    pltpu.sync_copy(m_hbm, m_v)
    pltpu.sync_copy(cards_hbm.at[wid], cards_v)
    lane = lax.iota(jnp.int32, _L)

    pre = [cards_v[n, pl.ds(0, _L)] * _DP2 for n in range(_N + 1)]
    srow = lane * _D

    @plsc.parallel_loop(0, _DP2, unroll=4, carry=jnp.zeros((_L,), jnp.int32))
    def _col(d, dv):
        g0 = plsc.bitcast(plsc.load_gather(m_v, [pre[0] + dv]), jnp.bfloat16)
        g1 = plsc.bitcast(plsc.load_gather(m_v, [pre[1] + dv]), jnp.bfloat16)
        g2 = plsc.bitcast(plsc.load_gather(m_v, [pre[2] + dv]), jnp.bfloat16)
        g3 = plsc.bitcast(plsc.load_gather(m_v, [pre[3] + dv]), jnp.bfloat16)
        g4 = plsc.bitcast(plsc.load_gather(m_v, [pre[4] + dv]), jnp.bfloat16)
        g5 = plsc.bitcast(plsc.load_gather(m_v, [pre[5] + dv]), jnp.bfloat16)
        g6 = plsc.bitcast(plsc.load_gather(m_v, [pre[6] + dv]), jnp.bfloat16)
        g7 = plsc.bitcast(plsc.load_gather(m_v, [pre[7] + dv]), jnp.bfloat16)
        acc = ((g0 + g1) + (g2 + g3)) + ((g4 + g5) + (g6 + g7))
        acc = jnp.maximum(acc, jnp.bfloat16(0.0))
        lo, hi = plsc.unpack(acc, format=plsc.PackFormat.INTERLEAVED,
                             preferred_element_type=jnp.float32)
        dv2 = dv + dv
        plsc.store_scatter(out_v, [srow + dv2], lo)
        plsc.store_scatter(out_v, [srow + dv2 + 1], hi)
        return dv + 1

    pltpu.sync_copy(out_v, out_hbm.at[pl.ds(wid * _BPW * _D, _BPW * _D)])


def _sc_call(cards_sc, m_pad):
    mesh = plsc.VectorSubcoreMesh(core_axis_name="c", subcore_axis_name="s")
    cp = pltpu.CompilerParams()
    if "needs_layout_passes" in pltpu.CompilerParams.__dataclass_fields__:
        cp = dataclasses.replace(cp, needs_layout_passes=False)
    run = pl.kernel(
        _sc_body,
        mesh=mesh,
        compiler_params=cp,
        out_type=jax.ShapeDtypeStruct((_RSC * _D,), jnp.float32),
        scratch_types=[
            pltpu.VMEM((_MR * _DP2,), jnp.int32),
            pltpu.VMEM((_N + 1, _BPW), jnp.int32),
            pltpu.VMEM((_BPW * _D,), jnp.float32),
        ],
    )
    m_pack = lax.bitcast_convert_type(
        m_pad.astype(jnp.bfloat16).reshape(_MR, _DP2, 2), jnp.int32)
    return run(cards_sc, m_pack.reshape(-1))


def _tc_kernel(cards_ref, rank_ref, suit_ref, card_ref, w_ref, b_ref,
               out_ref, m_scr):
    @pl.when(pl.program_id(0) == 0)
    def _build():
        m_scr[...] = _table_body(rank_ref, suit_ref, card_ref,
                                 w_ref).astype(jnp.bfloat16)

    cards = cards_ref[...].astype(jnp.bfloat16)
    bins = lax.broadcasted_iota(jnp.int32, (_TCBLK, _C), 1).astype(jnp.bfloat16)
    counts = jnp.zeros((_TCBLK, _C), jnp.bfloat16)
    for n in range(_N):
        counts += jnp.where(cards[:, n:n + 1] == bins,
                            jnp.bfloat16(1.0), jnp.bfloat16(0.0))
    acc = lax.dot_general(
        counts, m_scr[...], (((1,), (0,)), ((), ())),
        preferred_element_type=jnp.float32)
    out_ref[...] = jnp.maximum(acc + b_ref[...], 0.0)


def _tc_call(cards_tc, rank_emb, suit_emb, card_emb, W, b):
    n_rows = cards_tc.shape[0]
    grid = (n_rows // _TCBLK,)
    return pl.pallas_call(
        _tc_kernel,
        grid=grid,
        in_specs=[
            pl.BlockSpec((_TCBLK, _N), lambda i: (i, 0)),
            pl.BlockSpec((13, _D), lambda i: (0, 0)),
            pl.BlockSpec((4, _D), lambda i: (0, 0)),
            pl.BlockSpec((52, _D), lambda i: (0, 0)),
            pl.BlockSpec((_D, _D), lambda i: (0, 0)),
            pl.BlockSpec((1, _D), lambda i: (0, 0)),
        ],
        out_specs=pl.BlockSpec((_TCBLK, _D), lambda i: (i, 0)),
        out_shape=jax.ShapeDtypeStruct((n_rows, _D), jnp.float32),
        scratch_shapes=[pltpu.VMEM((_C, _D), jnp.bfloat16)],
    )(cards_tc, rank_emb, suit_emb, card_emb, W, b.reshape(1, _D))


def kernel(cards, rank_emb, suit_emb, card_emb, W, b):
    tc_out = _tc_call(cards, rank_emb, suit_emb, card_emb, W, b)
    m_pad = _build_table(rank_emb, suit_emb, card_emb, W, b)
    cards_t = cards[:_RSC].reshape(_NW, _BPW, _N).transpose(0, 2, 1)
    bias_row = jnp.full((_NW, 1, _BPW), _C, jnp.int32)
    cards_sc = jnp.concatenate([cards_t, bias_row], axis=1)
    sc_out = _sc_call(cards_sc, m_pad).reshape(_RSC, _D)
    return lax.dynamic_update_slice(tc_out, sc_out, (0, 0))
